# Initial kernel scaffold; baseline (speedup 1.0000x reference)
import jax
import jax.numpy as jnp
from jax.experimental import pallas as pl


def kernel(x):
    raise NotImplementedError("write your pallas kernel here")


# speedup vs baseline: 2.6018x; 2.6018x over previous
import jax
import jax.numpy as jnp
import numpy as np
from jax import lax
from jax.experimental import pallas as pl
from jax.experimental.pallas import tpu as pltpu, tpu_sc as plsc

R, N, TOPK = 64, 32768, 256
L = 16
NC, NS = 2, 16
NW = NC * NS
ROWS_PER_W = R // NW
CH = N // L

_SIGN = np.uint32(0x80000000)
_ONES = np.uint32(0xFFFFFFFF)


def _keys(xv):
    u = lax.bitcast_convert_type(xv, jnp.uint32)
    return u ^ jnp.where(u >= _SIGN, _ONES, _SIGN)


def _body(x_hbm, out_hbm, row_v, key_v):
    wid = lax.axis_index("s") * NC + lax.axis_index("c")
    for r in range(ROWS_PER_W):
        row = wid * ROWS_PER_W + r
        pltpu.sync_copy(x_hbm.at[row], row_v)

        def to_keys(j, _):
            sl = pl.ds(j * L, L)
            key_v[sl] = _keys(row_v[sl])
            return 0

        lax.fori_loop(0, CH, to_keys, 0)

        def count_ge(cand):
            def cbody(j, acc):
                kv = key_v[pl.ds(j * L, L)]
                return acc + plsc.all_reduce_population_count(kv >= cand)

            return lax.fori_loop(0, CH, cbody, jnp.zeros((L,), jnp.int32))

        def bit_body(t, prefix):
            bit = lax.shift_right_logical(
                jnp.full((L,), _SIGN, jnp.uint32),
                lax.broadcast(t.astype(jnp.uint32), (L,)))
            cand = prefix | bit
            return jnp.where(count_ge(cand) >= TOPK, cand, prefix)

        thresh = lax.fori_loop(0, 32, bit_body, jnp.zeros((L,), jnp.uint32))

        def mask_body(j, _):
            sl = pl.ds(j * L, L)
            keep = key_v[sl] >= thresh
            row_v[sl] = jnp.where(keep, row_v[sl], 0.0)
            return 0

        lax.fori_loop(0, CH, mask_body, 0)

        pltpu.sync_copy(row_v, out_hbm.at[row])


@jax.jit
def kernel(x):
    mesh = plsc.VectorSubcoreMesh(
        core_axis_name="c", subcore_axis_name="s",
        num_cores=NC, num_subcores=NS)
    return pl.kernel(
        _body,
        out_type=jax.ShapeDtypeStruct((R, N), jnp.float32),
        mesh=mesh,
        compiler_params=pltpu.CompilerParams(needs_layout_passes=False),
        scratch_types=[
            pltpu.VMEM((N,), jnp.float32),
            pltpu.VMEM((N,), jnp.uint32),
        ],
    )(x)


# speedup vs baseline: 12.2159x; 4.6951x over previous
import jax
import jax.numpy as jnp
import numpy as np
from jax import lax
from jax.experimental import pallas as pl
from jax.experimental.pallas import tpu as pltpu, tpu_sc as plsc

R, N, TOPK = 64, 32768, 256
L = 16
NC, NS = 2, 16
NW = NC * NS
ROWS_PER_W = R // NW
CH = N // L
HB = 256

_SIGN = np.uint32(0x80000000)
_ONES = np.uint32(0xFFFFFFFF)


def _keys(xv):
    u = lax.bitcast_convert_type(xv, jnp.uint32)
    return u ^ jnp.where(u >= _SIGN, _ONES, _SIGN)


def _body(x_hbm, out_hbm, row_v, key_v, cand_v, hist_v):
    wid = lax.axis_index("s") * NC + lax.axis_index("c")
    iota = lax.iota(jnp.int32, L)
    ones = jnp.ones((L,), jnp.int32)
    zeros = jnp.zeros((L,), jnp.int32)

    def zero_hist():
        def zh(i, _):
            hist_v[pl.ds(i * L, L)] = zeros
            return 0
        lax.fori_loop(0, HB // L, zh, 0)

    for r in range(ROWS_PER_W):
        row = wid * ROWS_PER_W + r
        pltpu.sync_copy(x_hbm.at[row], row_v)

        zero_hist()

        def pass_a(j, _):
            sl = pl.ds(j * L, L)
            kv = _keys(row_v[sl])
            key_v[sl] = kv
            digit = lax.shift_right_logical(kv, jnp.broadcast_to(np.uint32(28), (L,)))
            idx = lax.bitcast_convert_type(digit, jnp.int32) * L + iota
            plsc.addupdate_scatter(hist_v, [idx], ones)
            return 0

        lax.fori_loop(0, CH, pass_a, 0)

        prefix = np.uint32(0)
        kk = np.int32(TOPK)
        n_cand = None

        for l in range(8):
            def scan_body(i, c, kk=kk):
                cum, dstar, knext = c
                d = 15 - i
                tot = jnp.sum(hist_v[pl.ds(d * L, L)])
                cum2 = cum + tot
                hit = (cum2 >= kk) & (cum < kk)
                dstar = jnp.where(hit, d, dstar)
                knext = jnp.where(hit, kk - cum, knext)
                return (cum2, dstar, knext)

            _, dstar, knext = lax.fori_loop(
                0, 16, scan_body,
                (np.int32(0), np.int32(0), np.int32(0)))
            dstar_u = lax.convert_element_type(dstar, jnp.uint32)
            prefix = prefix | lax.shift_left(dstar_u, np.uint32(28 - 4 * l))

            if l == 7:
                break

            zero_hist()
            sh_this = np.uint32(28 - 4 * l)
            sh_next = np.uint32(28 - 4 * (l + 1))
            src = key_v if l == 0 else cand_v

            def cbody(j, off, sh_this=sh_this, sh_next=sh_next, src=src,
                      dstar_u=dstar_u, lvl=l, n_src=n_cand):
                kv = src[pl.ds(j * L, L)]
                digit = jnp.bitwise_and(
                    jnp.right_shift(kv, sh_this), np.uint32(15))
                m = digit == dstar_u
                if lvl > 0:
                    m = m & ((j * L + iota) < n_src)
                plsc.store_compressed(cand_v.at[pl.ds(off, L)], kv, mask=m)
                nd = jnp.bitwise_and(
                    jnp.right_shift(kv, sh_next), np.uint32(15))
                idx = lax.bitcast_convert_type(nd, jnp.int32) * L + iota
                plsc.addupdate_scatter(hist_v, [idx], ones, mask=m)
                cnt = plsc.all_reduce_population_count(m)
                return off + cnt[0]

            trip = CH if l == 0 else (n_cand + (L - 1)) // L
            n_cand = lax.fori_loop(0, trip, cbody, np.int32(0))
            kk = knext

        thresh = prefix

        def mask_body(j, _):
            sl = pl.ds(j * L, L)
            keep = key_v[sl] >= thresh
            row_v[sl] = jnp.where(keep, row_v[sl], 0.0)
            return 0

        lax.fori_loop(0, CH, mask_body, 0)

        pltpu.sync_copy(row_v, out_hbm.at[row])


@jax.jit
def kernel(x):
    mesh = plsc.VectorSubcoreMesh(
        core_axis_name="c", subcore_axis_name="s",
        num_cores=NC, num_subcores=NS)
    return pl.kernel(
        _body,
        out_type=jax.ShapeDtypeStruct((R, N), jnp.float32),
        mesh=mesh,
        compiler_params=pltpu.CompilerParams(needs_layout_passes=False),
        scratch_types=[
            pltpu.VMEM((N,), jnp.float32),
            pltpu.VMEM((N,), jnp.uint32),
            pltpu.VMEM((N,), jnp.uint32),
            pltpu.VMEM((HB,), jnp.int32),
        ],
    )(x)


# speedup vs baseline: 14.3232x; 1.1725x over previous
import jax
import jax.numpy as jnp
import numpy as np
from jax import lax
from jax.experimental import pallas as pl
from jax.experimental.pallas import tpu as pltpu, tpu_sc as plsc

R, N, TOPK = 64, 32768, 256
L = 16
NC, NS = 2, 16
NW = NC * NS
ROWS_PER_W = R // NW
CH = N // L
HB = 256
UNROLL = 8

_SIGN = np.uint32(0x80000000)
_ONES = np.uint32(0xFFFFFFFF)


def _keys(xv):
    u = lax.bitcast_convert_type(xv, jnp.uint32)
    return u ^ jnp.where(u >= _SIGN, _ONES, _SIGN)


def _process_row(row_v, cand_v, hist_v):
    iota = lax.iota(jnp.int32, L)
    iota16 = iota * L
    ones = jnp.ones((L,), jnp.int32)
    zeros = jnp.zeros((L,), jnp.int32)

    def zero_hist():
        def zh(i, _):
            hist_v[pl.ds(i * L, L)] = zeros
            return 0
        lax.fori_loop(0, HB // L, zh, 0, unroll=4)

    zero_hist()

    def pass_a(j, _):
        kv = _keys(row_v[pl.ds(j * L, L)])
        digit = lax.shift_right_logical(kv, jnp.broadcast_to(np.uint32(28), (L,)))
        idx = iota16 + lax.bitcast_convert_type(digit, jnp.int32)
        plsc.addupdate_scatter(hist_v, [idx], ones)
        return 0

    lax.fori_loop(0, CH, pass_a, 0, unroll=UNROLL)

    prefix = np.uint32(0)
    kk = np.int32(TOPK)
    n_cand = None

    for l in range(8):
        tot = hist_v[pl.ds(0, L)]
        for i in range(1, L):
            tot = tot + hist_v[pl.ds(i * L, L)]
        cum_ge = lax.rev(plsc.cumsum(lax.rev(tot, (0,))), (0,))
        sel = cum_ge >= kk
        dstar = jnp.max(jnp.where(sel, iota, -1))
        knext = jnp.max(jnp.where(sel, kk - (cum_ge - tot), np.int32(-(2**31))))
        dstar_u = lax.convert_element_type(dstar, jnp.uint32)
        prefix = prefix | lax.shift_left(dstar_u, np.uint32(28 - 4 * l))

        if l == 7:
            break

        zero_hist()
        sh_this = np.uint32(28 - 4 * l)
        sh_next = np.uint32(28 - 4 * (l + 1))

        def cbody(j, off, sh_this=sh_this, sh_next=sh_next,
                  dstar_u=dstar_u, lvl=l, n_src=n_cand):
            if lvl == 0:
                kv = _keys(row_v[pl.ds(j * L, L)])
            else:
                kv = cand_v[pl.ds(j * L, L)]
            digit = jnp.bitwise_and(
                jnp.right_shift(kv, sh_this), np.uint32(15))
            m = digit == dstar_u
            if lvl > 0:
                m = m & ((j * L + iota) < n_src)
            plsc.store_compressed(cand_v.at[pl.ds(off, L)], kv, mask=m)
            nd = jnp.bitwise_and(
                jnp.right_shift(kv, sh_next), np.uint32(15))
            idx = iota16 + lax.bitcast_convert_type(nd, jnp.int32)
            plsc.addupdate_scatter(hist_v, [idx], ones, mask=m)
            cnt = plsc.all_reduce_population_count(m)
            return off + cnt[0]

        if l == 0:
            n_cand = lax.fori_loop(0, CH, cbody, np.int32(0), unroll=UNROLL)
        else:
            trip = (n_cand + (L - 1)) // L
            n_cand = lax.fori_loop(0, trip, cbody, np.int32(0))
        kk = knext

    thresh = prefix

    def mask_body(j, _):
        sl = pl.ds(j * L, L)
        xv = row_v[sl]
        keep = _keys(xv) >= thresh
        row_v[sl] = jnp.where(keep, xv, 0.0)
        return 0

    lax.fori_loop(0, CH, mask_body, 0, unroll=UNROLL)


def _body(x_hbm, out_hbm, row_a, row_b, cand_v, hist_v, sem_a, sem_b):
    wid = lax.axis_index("s") * NC + lax.axis_index("c")
    row0 = wid * ROWS_PER_W
    row1 = row0 + 1

    in0 = pltpu.async_copy(x_hbm.at[row0], row_a, sem_a)
    in1 = pltpu.async_copy(x_hbm.at[row1], row_b, sem_b)
    in0.wait()
    _process_row(row_a, cand_v, hist_v)
    out0 = pltpu.async_copy(row_a, out_hbm.at[row0], sem_a)
    in1.wait()
    _process_row(row_b, cand_v, hist_v)
    out0.wait()
    pltpu.async_copy(row_b, out_hbm.at[row1], sem_b).wait()


@jax.jit
def kernel(x):
    mesh = plsc.VectorSubcoreMesh(
        core_axis_name="c", subcore_axis_name="s",
        num_cores=NC, num_subcores=NS)
    return pl.kernel(
        _body,
        out_type=jax.ShapeDtypeStruct((R, N), jnp.float32),
        mesh=mesh,
        compiler_params=pltpu.CompilerParams(needs_layout_passes=False),
        scratch_types=[
            pltpu.VMEM((N,), jnp.float32),
            pltpu.VMEM((N,), jnp.float32),
            pltpu.VMEM((N,), jnp.uint32),
            pltpu.VMEM((HB,), jnp.int32),
            pltpu.SemaphoreType.DMA,
            pltpu.SemaphoreType.DMA,
        ],
    )(x)


# speedup vs baseline: 19.3815x; 1.3532x over previous
import jax
import jax.numpy as jnp
import numpy as np
from jax import lax
from jax.experimental import pallas as pl
from jax.experimental.pallas import tpu as pltpu, tpu_sc as plsc

R, N, TOPK = 64, 32768, 256
L = 16
NC, NS = 2, 16
NW = NC * NS
ROWS_PER_W = R // NW
CH = N // L
HB = 256
UNROLL = 8

_SIGN = np.uint32(0x80000000)
_ONES = np.uint32(0xFFFFFFFF)


def _keys(xv):
    u = lax.bitcast_convert_type(xv, jnp.uint32)
    return u ^ jnp.where(u >= _SIGN, _ONES, _SIGN)


def _process_row(row_v, cand_v, hist_v):
    iota = lax.iota(jnp.int32, L)
    iota16 = iota * L
    ones = jnp.ones((L,), jnp.int32)
    zeros = jnp.zeros((L,), jnp.int32)

    def zero_hist():
        def zh(i, _):
            hist_v[pl.ds(i * L, L)] = zeros
            return 0
        lax.fori_loop(0, HB // L, zh, 0, unroll=4)

    zero_hist()

    @plsc.parallel_loop(0, CH, unroll=UNROLL)
    def _(j):
        kv = _keys(row_v[pl.ds(j * L, L)])
        digit = lax.shift_right_logical(kv, jnp.broadcast_to(np.uint32(28), (L,)))
        idx = iota16 + lax.bitcast_convert_type(digit, jnp.int32)
        plsc.addupdate_scatter(hist_v, [idx], ones)

    prefix = np.uint32(0)
    kk = np.int32(TOPK)
    n_cand = None

    for l in range(8):
        tot = hist_v[pl.ds(0, L)]
        for i in range(1, L):
            tot = tot + hist_v[pl.ds(i * L, L)]
        cum_ge = lax.rev(plsc.cumsum(lax.rev(tot, (0,))), (0,))
        sel = cum_ge >= kk
        dstar = jnp.max(jnp.where(sel, iota, -1))
        knext = jnp.max(jnp.where(sel, kk - (cum_ge - tot), np.int32(-(2**31))))
        dstar_u = lax.convert_element_type(dstar, jnp.uint32)
        prefix = prefix | lax.shift_left(dstar_u, np.uint32(28 - 4 * l))

        if l == 7:
            break

        zero_hist()
        sh_this = np.uint32(28 - 4 * l)
        sh_next = np.uint32(28 - 4 * (l + 1))

        def cbody(j, off, sh_this=sh_this, sh_next=sh_next,
                  dstar_u=dstar_u, lvl=l, n_src=n_cand):
            if lvl == 0:
                kv = _keys(row_v[pl.ds(j * L, L)])
            else:
                kv = cand_v[pl.ds(j * L, L)]
            digit = jnp.bitwise_and(
                jnp.right_shift(kv, sh_this), np.uint32(15))
            m = digit == dstar_u
            if lvl > 0:
                m = m & ((j * L + iota) < n_src)
            plsc.store_compressed(cand_v.at[pl.ds(off, L)], kv, mask=m)
            nd = jnp.bitwise_and(
                jnp.right_shift(kv, sh_next), np.uint32(15))
            idx = iota16 + lax.bitcast_convert_type(nd, jnp.int32)
            plsc.addupdate_scatter(hist_v, [idx], ones, mask=m)
            cnt = plsc.all_reduce_population_count(m)
            return off + cnt[0]

        if l == 0:
            n_cand = lax.fori_loop(0, CH, cbody, np.int32(0), unroll=UNROLL)
        else:
            trip = (n_cand + (L - 1)) // L
            n_cand = lax.fori_loop(0, trip, cbody, np.int32(0))
        kk = knext

    thresh = prefix

    @plsc.parallel_loop(0, CH, unroll=UNROLL)
    def _(j):
        sl = pl.ds(j * L, L)
        xv = row_v[sl]
        keep = _keys(xv) >= thresh
        row_v[sl] = jnp.where(keep, xv, 0.0)


def _body(x_hbm, out_hbm, row_a, row_b, cand_v, hist_v, sem_a, sem_b):
    wid = lax.axis_index("s") * NC + lax.axis_index("c")
    row0 = wid * ROWS_PER_W
    row1 = row0 + 1

    in0 = pltpu.async_copy(x_hbm.at[row0], row_a, sem_a)
    in1 = pltpu.async_copy(x_hbm.at[row1], row_b, sem_b)
    in0.wait()
    _process_row(row_a, cand_v, hist_v)
    out0 = pltpu.async_copy(row_a, out_hbm.at[row0], sem_a)
    in1.wait()
    _process_row(row_b, cand_v, hist_v)
    out0.wait()
    pltpu.async_copy(row_b, out_hbm.at[row1], sem_b).wait()


@jax.jit
def kernel(x):
    mesh = plsc.VectorSubcoreMesh(
        core_axis_name="c", subcore_axis_name="s",
        num_cores=NC, num_subcores=NS)
    return pl.kernel(
        _body,
        out_type=jax.ShapeDtypeStruct((R, N), jnp.float32),
        mesh=mesh,
        compiler_params=pltpu.CompilerParams(needs_layout_passes=False),
        scratch_types=[
            pltpu.VMEM((N,), jnp.float32),
            pltpu.VMEM((N,), jnp.float32),
            pltpu.VMEM((N,), jnp.uint32),
            pltpu.VMEM((HB,), jnp.int32),
            pltpu.SemaphoreType.DMA,
            pltpu.SemaphoreType.DMA,
        ],
    )(x)


# speedup vs baseline: 30.5768x; 1.5776x over previous
import jax
import jax.numpy as jnp
import numpy as np
from jax import lax
from jax.experimental import pallas as pl
from jax.experimental.pallas import tpu as pltpu, tpu_sc as plsc

R, N, TOPK = 64, 32768, 256
L = 16
NC, NS = 2, 16
NW = NC * NS
ROWS_PER_W = R // NW
CH = N // L
HB = 256
UNROLL = 8

_SIGN = np.uint32(0x80000000)
_ONES = np.uint32(0xFFFFFFFF)


def _keys(xv):
    u = lax.bitcast_convert_type(xv, jnp.uint32)
    return u ^ jnp.where(u >= _SIGN, _ONES, _SIGN)


def _process_row(row_v, cand_v, hist_v):
    iota = lax.iota(jnp.int32, L)
    iota16 = iota * L
    ones = jnp.ones((L,), jnp.int32)
    zeros = jnp.zeros((L,), jnp.int32)

    def zero_hist():
        def zh(i, _):
            hist_v[pl.ds(i * L, L)] = zeros
            return 0
        lax.fori_loop(0, HB // L, zh, 0, unroll=4)

    zero_hist()

    @plsc.parallel_loop(0, CH, unroll=UNROLL)
    def _(j):
        kv = _keys(row_v[pl.ds(j * L, L)])
        digit = lax.shift_right_logical(kv, jnp.broadcast_to(np.uint32(28), (L,)))
        idx = iota16 + lax.bitcast_convert_type(digit, jnp.int32)
        plsc.addupdate_scatter(hist_v, [idx], ones)

    prefix = np.uint32(0)
    kk = np.int32(TOPK)
    n_cand = None

    for l in range(8):
        tot = hist_v[pl.ds(0, L)]
        for i in range(1, L):
            tot = tot + hist_v[pl.ds(i * L, L)]
        cum_ge = lax.rev(plsc.cumsum(lax.rev(tot, (0,))), (0,))
        sel = cum_ge >= kk
        dstar = jnp.max(jnp.where(sel, iota, -1))
        knext = jnp.max(jnp.where(sel, kk - (cum_ge - tot), np.int32(-(2**31))))
        dstar_u = lax.convert_element_type(dstar, jnp.uint32)
        prefix = prefix | lax.shift_left(dstar_u, np.uint32(28 - 4 * l))

        if l == 7:
            break

        zero_hist()
        sh_this = np.uint32(28 - 4 * l)
        sh_next = np.uint32(28 - 4 * (l + 1))

        def cbody(j, off, sh_this=sh_this, sh_next=sh_next,
                  dstar_u=dstar_u, lvl=l, n_src=n_cand):
            if lvl == 0:
                kv = _keys(row_v[pl.ds(j * L, L)])
            else:
                kv = cand_v[pl.ds(j * L, L)]
            digit = jnp.bitwise_and(
                jnp.right_shift(kv, sh_this), np.uint32(15))
            m = digit == dstar_u
            if lvl > 0:
                m = m & ((j * L + iota) < n_src)
            plsc.store_compressed(cand_v.at[pl.ds(off, L)], kv, mask=m)
            nd = jnp.bitwise_and(
                jnp.right_shift(kv, sh_next), np.uint32(15))
            idx = iota16 + lax.bitcast_convert_type(nd, jnp.int32)
            plsc.addupdate_scatter(hist_v, [idx], ones, mask=m)
            cnt = plsc.all_reduce_population_count(m)
            return off + cnt[0]

        if l == 0:
            def cbody_pl(j, off, cbody=cbody):
                return cbody(j, off)
            n_cand = plsc.parallel_loop(
                0, CH, unroll=UNROLL, carry=dstar * 0)(cbody_pl)
        else:
            trip = (n_cand + (L - 1)) // L
            n_cand = lax.fori_loop(0, trip, cbody, np.int32(0))
        kk = knext

    thresh = prefix

    @plsc.parallel_loop(0, CH, unroll=UNROLL)
    def _(j):
        sl = pl.ds(j * L, L)
        xv = row_v[sl]
        keep = _keys(xv) >= thresh
        row_v[sl] = jnp.where(keep, xv, 0.0)


def _body(x_hbm, out_hbm, row_a, row_b, cand_v, hist_v, sem_a, sem_b):
    wid = lax.axis_index("s") * NC + lax.axis_index("c")
    row0 = wid * ROWS_PER_W
    row1 = row0 + 1

    in0 = pltpu.async_copy(x_hbm.at[row0], row_a, sem_a)
    in1 = pltpu.async_copy(x_hbm.at[row1], row_b, sem_b)
    in0.wait()
    _process_row(row_a, cand_v, hist_v)
    out0 = pltpu.async_copy(row_a, out_hbm.at[row0], sem_a)
    in1.wait()
    _process_row(row_b, cand_v, hist_v)
    out0.wait()
    pltpu.async_copy(row_b, out_hbm.at[row1], sem_b).wait()


@jax.jit
def kernel(x):
    mesh = plsc.VectorSubcoreMesh(
        core_axis_name="c", subcore_axis_name="s",
        num_cores=NC, num_subcores=NS)
    return pl.kernel(
        _body,
        out_type=jax.ShapeDtypeStruct((R, N), jnp.float32),
        mesh=mesh,
        compiler_params=pltpu.CompilerParams(needs_layout_passes=False),
        scratch_types=[
            pltpu.VMEM((N,), jnp.float32),
            pltpu.VMEM((N,), jnp.float32),
            pltpu.VMEM((N,), jnp.uint32),
            pltpu.VMEM((HB,), jnp.int32),
            pltpu.SemaphoreType.DMA,
            pltpu.SemaphoreType.DMA,
        ],
    )(x)
